# baseline (speedup 1.0000x reference)
import functools

import jax
import jax.numpy as jnp
from jax import lax
from jax.experimental import pallas as pl
from jax.experimental.pallas import tpu as pltpu
from jax.experimental.pallas import tpu_sc as plsc

_D = 2048
_K = 307
_NC = 2
_NS = 16
_NW = _NC * _NS
_CH = 16
_NB1 = 1024
_NBS = 128
_CAP = 1008


def _sc_gating_mask(xf):
    R = xf.shape[0]
    rows_per_w = R // _NW
    nch = rows_per_w // _CH

    mesh = plsc.VectorSubcoreMesh(core_axis_name="c", subcore_axis_name="s")

    @functools.partial(
        pl.kernel,
        mesh=mesh,
        compiler_params=pltpu.CompilerParams(
            needs_layout_passes=False, use_tc_tiling_on_sc=True
        ),
        out_type=jax.ShapeDtypeStruct((R, _D), jnp.float32),
        scratch_types=[
            pltpu.VMEM((_CH, _D), jnp.float32),
            pltpu.VMEM((_CH, _D), jnp.float32),
            pltpu.VMEM((_CH, _D), jnp.float32),
            pltpu.VMEM((_NB1 * 16,), jnp.int32),
            pltpu.VMEM((_CAP * 16,), jnp.int32),
            pltpu.VMEM((16,), jnp.int32),
            pltpu.SemaphoreType.DMA,
            pltpu.SemaphoreType.DMA,
            pltpu.SemaphoreType.DMA,
        ],
    )
    def k(x_hbm, m_hbm, in0, in1, outb, hist, cand, tref, s0, s1, s_out):
        wid = lax.axis_index("c") * _NS + lax.axis_index("s")
        rbase = wid * rows_per_w
        lane = lax.iota(jnp.int32, 16)
        ones = jnp.full((16,), 1, jnp.int32)
        zeros = jnp.zeros((16,), jnp.int32)
        kvec = jnp.full((16,), _K, jnp.int32)

        def roff(i):
            return rbase + i * _CH

        def start_in(i, buf, sem):
            pltpu.make_async_copy(
                x_hbm.at[pl.ds(roff(i), _CH), :], buf, sem
            ).start()

        def wait_in(i, buf, sem):
            pltpu.make_async_copy(
                x_hbm.at[pl.ds(roff(i), _CH), :], buf, sem
            ).wait()

        @plsc.parallel_loop(0, _NB1, unroll=4)
        def _(j):
            hist[pl.ds(j * 16, 16)] = zeros

        def scan(nb, kk):
            @plsc.parallel_loop(0, nb, unroll=4, carry=(zeros, zeros, zeros))
            def res(j, carry):
                acc, nc_cnt, cnt_above = carry
                beta = nb - 1 - j
                v = hist[pl.ds(beta * 16, 16)]
                hist[pl.ds(beta * 16, 16)] = zeros
                acc = acc + v
                nc = acc < kk
                nc_cnt = nc_cnt + jnp.where(nc, 1, 0)
                cnt_above = jnp.where(nc, acc, cnt_above)
                return acc, nc_cnt, cnt_above

            _, nc_cnt, cnt_above = res
            return (nb - 1) - nc_cnt, kk - cnt_above

        def gat_u(buf, d):
            v = plsc.load_gather(buf, [lane, d ^ lane])
            return lax.bitcast_convert_type(v, jnp.int32) & jnp.int32(
                0x7FFFFFFF
            )

        def refine(kk1, b1, loop_hi, load_u, valid_fn):
            @plsc.parallel_loop(0, loop_hi, unroll=4)
            def _(j):
                u = load_u(j)
                m = valid_fn(j, u >> 21, b1)
                addr = ((u >> 10) & jnp.int32(0x7F0)) | lane
                plsc.addupdate_scatter(hist, [addr], ones, mask=m)

            b2, kk2 = scan(_NBS, kk1)
            pfx = (b1 << 7) | b2

            @plsc.parallel_loop(0, loop_hi, unroll=4)
            def _(j):
                u = load_u(j)
                m = valid_fn(j, u >> 14, pfx)
                addr = ((u >> 3) & jnp.int32(0x7F0)) | lane
                plsc.addupdate_scatter(hist, [addr], ones, mask=m)

            b3, kk3 = scan(_NBS, kk2)
            pfx = (pfx << 7) | b3

            @plsc.parallel_loop(0, loop_hi, unroll=4)
            def _(j):
                u = load_u(j)
                m = valid_fn(j, u >> 7, pfx)
                addr = ((u << 4) & jnp.int32(0x7F0)) | lane
                plsc.addupdate_scatter(hist, [addr], ones, mask=m)

            b4, _ = scan(_NBS, kk3)
            tref[...] = (pfx << 7) | b4

        def process(i, buf, pbuf, sem, psem):
            wait_in(i, buf, sem)

            @pl.when(i > 1)
            def _():
                pltpu.make_async_copy(
                    outb, m_hbm.at[pl.ds(roff(i) - 2 * _CH, _CH), :], s_out
                ).wait()

            tvec = tref[...]

            @plsc.parallel_loop(0, _D, unroll=8)
            def _(d):
                up = gat_u(pbuf, d)
                m = jnp.where(up >= tvec, 1.0, 0.0).astype(jnp.float32)
                plsc.store_scatter(outb, [lane, d ^ lane], m)

            @plsc.parallel_loop(0, _D, unroll=8)
            def _(d):
                u = gat_u(buf, d)
                addr = ((u >> 17) & jnp.int32(0x3FF0)) | lane
                plsc.addupdate_scatter(hist, [addr], ones)

            @pl.when(i > 0)
            def _():
                pltpu.make_async_copy(
                    outb, m_hbm.at[pl.ds(roff(i) - _CH, _CH), :], s_out
                ).start()

            @pl.when(i + 1 < nch)
            def _():
                start_in(i + 1, pbuf, psem)

            b1, kk1 = scan(_NB1, kvec)

            @plsc.parallel_loop(0, _D, unroll=8, carry=zeros)
            def cnt(d, c):
                u = gat_u(buf, d)
                m = ((u >> 21) == b1) & (c < _CAP)
                plsc.store_scatter(cand, [(c << 4) | lane], u, mask=m)
                return c + jnp.where(m, 1, 0)

            m_max = jnp.max(cnt)

            refine(
                kk1,
                b1,
                m_max,
                lambda j: cand[pl.ds(j * 16, 16)],
                lambda j, upart, pfx: (j < cnt) & (upart == pfx),
            )

            @pl.when(m_max >= _CAP)
            def _():
                refine(
                    kk1,
                    b1,
                    _D,
                    lambda d: gat_u(buf, d),
                    lambda d, upart, pfx: upart == pfx,
                )

        start_in(0, in0, s0)

        def pair(p, c):
            process(p * 2, in0, in1, s0, s1)
            process(p * 2 + 1, in1, in0, s1, s0)
            return c

        lax.fori_loop(0, nch // 2, pair, 0)

        pltpu.make_async_copy(
            outb, m_hbm.at[pl.ds(roff(nch - 2), _CH), :], s_out
        ).wait()
        tvec = tref[...]
        lbuf = in1 if (nch - 1) % 2 else in0

        @plsc.parallel_loop(0, _D, unroll=8)
        def _(d):
            u = gat_u(lbuf, d)
            m = jnp.where(u >= tvec, 1.0, 0.0).astype(jnp.float32)
            plsc.store_scatter(outb, [lane, d ^ lane], m)

        pltpu.make_async_copy(
            outb, m_hbm.at[pl.ds(roff(nch - 1), _CH), :], s_out
        ).start()
        pltpu.make_async_copy(
            outb, m_hbm.at[pl.ds(roff(nch - 1), _CH), :], s_out
        ).wait()

    return k(xf)


def kernel(x):
    B, T, D = x.shape
    xf = x.reshape(B * T, D)
    mask = _sc_gating_mask(xf)
    return x, mask.reshape(B, T, D)


# speedup vs baseline: 1.1793x; 1.1793x over previous
import functools

import jax
import jax.numpy as jnp
from jax import lax
from jax.experimental import pallas as pl
from jax.experimental.pallas import tpu as pltpu
from jax.experimental.pallas import tpu_sc as plsc

_D = 2048
_K = 307
_NC = 2
_NS = 16
_NW = _NC * _NS
_CH = 16
_NB1 = 1024
_NBS = 128
_CAP = 1008


def _sc_gating_mask(xf):
    R = xf.shape[0]
    rows_per_w = R // _NW
    nch = rows_per_w // _CH

    mesh = plsc.VectorSubcoreMesh(core_axis_name="c", subcore_axis_name="s")

    @functools.partial(
        pl.kernel,
        mesh=mesh,
        compiler_params=pltpu.CompilerParams(
            needs_layout_passes=False, use_tc_tiling_on_sc=True
        ),
        out_type=jax.ShapeDtypeStruct((R, _D), jnp.float32),
        scratch_types=[
            pltpu.VMEM((_CH, _D), jnp.float32),
            pltpu.VMEM((_CH, _D), jnp.float32),
            pltpu.VMEM((_CH, _D), jnp.float32),
            pltpu.VMEM((_NB1 * 16,), jnp.int32),
            pltpu.VMEM((_CAP * 16,), jnp.int32),
            pltpu.VMEM((16,), jnp.int32),
            pltpu.SemaphoreType.DMA,
            pltpu.SemaphoreType.DMA,
            pltpu.SemaphoreType.DMA,
        ],
    )
    def k(x_hbm, m_hbm, in0, in1, outb, hist, cand, tref, s0, s1, s_out):
        wid = lax.axis_index("c") * _NS + lax.axis_index("s")
        rbase = wid * rows_per_w
        lane = lax.iota(jnp.int32, 16)
        ones = jnp.full((16,), 1, jnp.int32)
        zeros = jnp.zeros((16,), jnp.int32)
        kvec = jnp.full((16,), _K, jnp.int32)

        def roff(i):
            return rbase + i * _CH

        def start_in(i, buf, sem):
            pltpu.make_async_copy(
                x_hbm.at[pl.ds(roff(i), _CH), :], buf, sem
            ).start()

        def wait_in(i, buf, sem):
            pltpu.make_async_copy(
                x_hbm.at[pl.ds(roff(i), _CH), :], buf, sem
            ).wait()

        @plsc.parallel_loop(0, _NB1, unroll=4)
        def _(j):
            hist[pl.ds(j * 16, 16)] = zeros

        def scan(nb, kk):
            @plsc.parallel_loop(0, nb, unroll=4, carry=(zeros, zeros, zeros))
            def res(j, carry):
                acc, nc_cnt, cnt_above = carry
                beta = nb - 1 - j
                v = hist[pl.ds(beta * 16, 16)]
                hist[pl.ds(beta * 16, 16)] = zeros
                acc = acc + v
                nc = acc < kk
                nc_cnt = nc_cnt + jnp.where(nc, 1, 0)
                cnt_above = jnp.where(nc, acc, cnt_above)
                return acc, nc_cnt, cnt_above

            _, nc_cnt, cnt_above = res
            return (nb - 1) - nc_cnt, kk - cnt_above

        def gat_u(buf, d):
            v = plsc.load_gather(buf, [lane, d ^ lane])
            return lax.bitcast_convert_type(v, jnp.int32) & jnp.int32(
                0x7FFFFFFF
            )

        def refine(kk1, b1, loop_hi, load_u, valid_fn):
            @plsc.parallel_loop(0, loop_hi, unroll=4)
            def _(j):
                u = load_u(j)
                m = valid_fn(j, u >> 21, b1)
                addr = ((u >> 10) & jnp.int32(0x7F0)) | lane
                plsc.addupdate_scatter(hist, [addr], ones, mask=m)

            b2, kk2 = scan(_NBS, kk1)
            pfx = (b1 << 7) | b2

            @plsc.parallel_loop(0, loop_hi, unroll=4)
            def _(j):
                u = load_u(j)
                m = valid_fn(j, u >> 14, pfx)
                addr = ((u >> 3) & jnp.int32(0x7F0)) | lane
                plsc.addupdate_scatter(hist, [addr], ones, mask=m)

            b3, kk3 = scan(_NBS, kk2)
            pfx = (pfx << 7) | b3

            @plsc.parallel_loop(0, loop_hi, unroll=4)
            def _(j):
                u = load_u(j)
                m = valid_fn(j, u >> 7, pfx)
                addr = ((u << 4) & jnp.int32(0x7F0)) | lane
                plsc.addupdate_scatter(hist, [addr], ones, mask=m)

            b4, _ = scan(_NBS, kk3)
            tref[...] = (pfx << 7) | b4

        def process(i, buf, pbuf, sem, psem):
            wait_in(i, buf, sem)

            @pl.when(i > 1)
            def _():
                pltpu.make_async_copy(
                    outb, m_hbm.at[pl.ds(roff(i) - 2 * _CH, _CH), :], s_out
                ).wait()

            tvec = tref[...]

            @plsc.parallel_loop(0, _D, unroll=4)
            def _(d):
                col = d ^ lane
                vp = plsc.load_gather(pbuf, [lane, col])
                up = lax.bitcast_convert_type(vp, jnp.int32) & jnp.int32(
                    0x7FFFFFFF
                )
                m = jnp.where(up >= tvec, 1.0, 0.0).astype(jnp.float32)
                plsc.store_scatter(outb, [lane, col], m)
                v = plsc.load_gather(buf, [lane, col])
                u = lax.bitcast_convert_type(v, jnp.int32) & jnp.int32(
                    0x7FFFFFFF
                )
                addr = ((u >> 17) & jnp.int32(0x3FF0)) | lane
                plsc.addupdate_scatter(hist, [addr], ones)

            @pl.when(i > 0)
            def _():
                pltpu.make_async_copy(
                    outb, m_hbm.at[pl.ds(roff(i) - _CH, _CH), :], s_out
                ).start()

            @pl.when(i + 1 < nch)
            def _():
                start_in(i + 1, pbuf, psem)

            b1, kk1 = scan(_NB1, kvec)

            @plsc.parallel_loop(0, _D, unroll=8, carry=zeros)
            def cnt(d, c):
                u = gat_u(buf, d)
                m = ((u >> 21) == b1) & (c < _CAP)
                plsc.store_scatter(cand, [(c << 4) | lane], u, mask=m)
                return c + jnp.where(m, 1, 0)

            m_max = jnp.max(cnt)

            refine(
                kk1,
                b1,
                m_max,
                lambda j: cand[pl.ds(j * 16, 16)],
                lambda j, upart, pfx: (j < cnt) & (upart == pfx),
            )

            @pl.when(m_max >= _CAP)
            def _():
                refine(
                    kk1,
                    b1,
                    _D,
                    lambda d: gat_u(buf, d),
                    lambda d, upart, pfx: upart == pfx,
                )

        start_in(0, in0, s0)

        def pair(p, c):
            process(p * 2, in0, in1, s0, s1)
            process(p * 2 + 1, in1, in0, s1, s0)
            return c

        lax.fori_loop(0, nch // 2, pair, 0)

        pltpu.make_async_copy(
            outb, m_hbm.at[pl.ds(roff(nch - 2), _CH), :], s_out
        ).wait()
        tvec = tref[...]
        lbuf = in1 if (nch - 1) % 2 else in0

        @plsc.parallel_loop(0, _D, unroll=8)
        def _(d):
            u = gat_u(lbuf, d)
            m = jnp.where(u >= tvec, 1.0, 0.0).astype(jnp.float32)
            plsc.store_scatter(outb, [lane, d ^ lane], m)

        pltpu.make_async_copy(
            outb, m_hbm.at[pl.ds(roff(nch - 1), _CH), :], s_out
        ).start()
        pltpu.make_async_copy(
            outb, m_hbm.at[pl.ds(roff(nch - 1), _CH), :], s_out
        ).wait()

    return k(xf)


def kernel(x):
    B, T, D = x.shape
    xf = x.reshape(B * T, D)
    mask = _sc_gating_mask(xf)
    return x, mask.reshape(B, T, D)
